# baseline (speedup 1.0000x reference)
import functools

import jax
import jax.numpy as jnp
from jax import lax
from jax.experimental import pallas as pl
from jax.experimental.pallas import tpu as pltpu
from jax.experimental.pallas import tpu_sc as plsc

_BLK = 256


def _gate_body(xb_ref, gwt_ref, e0_ref, e1_ref, w0_ref, w1_ref):
    logits = lax.dot_general(xb_ref[...], gwt_ref[...], (((1,), (0,)), ((), ())),
                             preferred_element_type=jnp.float32)
    mx = jnp.max(logits, axis=1, keepdims=True)
    ex = jnp.exp(logits - mx)
    p = ex / jnp.sum(ex, axis=1, keepdims=True)
    tm, ne = p.shape
    a = p[:, :, None]
    b = p[:, None, :]
    ii = lax.broadcasted_iota(jnp.int32, (tm, ne, ne), 1)
    jj = lax.broadcasted_iota(jnp.int32, (tm, ne, ne), 2)
    beats = (b > a) | ((b == a) & (jj < ii))
    rank = jnp.sum(beats.astype(jnp.int32), axis=2)
    lane = lax.broadcasted_iota(jnp.int32, (tm, ne), 1)
    for k, (eref, wref) in enumerate(((e0_ref, w0_ref), (e1_ref, w1_ref))):
        sel = rank == k
        idx = jnp.sum(jnp.where(sel, lane, 0), axis=1)
        wv = jnp.sum(jnp.where(sel, p, 0.0), axis=1)
        eref[...] = idx.reshape(eref.shape)
        wref[...] = jnp.broadcast_to(wv[:, None], wref.shape)


def _disp_body(e0_ref, e1_ref, dest_ref, blk_ref, *, e_num):
    eid = jnp.concatenate([e0_ref[...], e1_ref[...]], axis=0)
    rows, lanes = eid.shape
    ci = lax.broadcasted_iota(jnp.int32, (lanes, lanes), 0)
    cj = lax.broadcasted_iota(jnp.int32, (lanes, lanes), 1)
    slt = (ci < cj).astype(jnp.bfloat16)
    ri = lax.broadcasted_iota(jnp.int32, (rows, rows), 0)
    rj = lax.broadcasted_iota(jnp.int32, (rows, rows), 1)
    ltr = (rj < ri).astype(jnp.bfloat16)
    ones = jnp.ones((lanes, lanes), jnp.bfloat16)
    dest = jnp.zeros((rows, lanes), jnp.float32)
    brows, blanes = blk_ref.shape
    bi = lax.broadcasted_iota(jnp.int32, (brows, blanes), 1).astype(jnp.float32)
    blkv = jnp.full((brows, blanes), -1.0, jnp.float32)
    rowbase = jnp.float32(0.0)
    blkbase = jnp.float32(0.0)
    for e in range(e_num):
        m = (eid == e).astype(jnp.bfloat16)
        lane_excl = lax.dot_general(m, slt, (((1,), (0,)), ((), ())),
                                    preferred_element_type=jnp.float32)
        rowsum = lax.dot_general(m, ones, (((1,), (0,)), ((), ())),
                                 preferred_element_type=jnp.float32)
        row_excl = lax.dot_general(ltr, rowsum.astype(jnp.bfloat16),
                                   (((1,), (0,)), ((), ())),
                                   preferred_element_type=jnp.float32)
        prefix = lane_excl + row_excl
        cnt = jnp.sum(rowsum[:, 0:1])
        dest = dest + jnp.where(eid == e, rowbase + prefix, 0.0)
        blkv = blkv + (bi >= blkbase).astype(jnp.float32)
        nb = jnp.ceil(cnt / _BLK)
        blkbase = blkbase + nb
        rowbase = rowbase + nb * _BLK
    dest_ref[...] = dest.astype(jnp.int32)
    blk_ref[...] = blkv.astype(jnp.int32)


def _permute_body(x3, tok, dst, xs_o, v_tok, v_dst, v_rows, sem,
                  *, slots_per, chunk):
    c = lax.axis_index("c")
    s = lax.axis_index("s")
    wid = s * 2 + c
    for ch in range(slots_per // chunk):
        base = wid * slots_per + ch * chunk
        pltpu.sync_copy(tok.at[pl.ds(base, chunk)], v_tok)
        pltpu.sync_copy(dst.at[pl.ds(base, chunk)], v_dst)
        pltpu.async_copy(x3.at[v_tok], v_rows, sem).wait()
        pltpu.async_copy(v_rows, xs_o.at[v_dst], sem).wait()


def _collect_body(ys3, dst, gsl_o, v_dst, v_rows, sem, *, slots_per, chunk):
    c = lax.axis_index("c")
    s = lax.axis_index("s")
    wid = s * 2 + c
    for ch in range(slots_per // chunk):
        base = wid * slots_per + ch * chunk
        pltpu.sync_copy(dst.at[pl.ds(base, chunk)], v_dst)
        pltpu.async_copy(ys3.at[v_dst], v_rows, sem).wait()
        pltpu.sync_copy(v_rows, gsl_o.at[pl.ds(base, chunk)])


def _gmm_body(blk_sm, xs_ref, g_ref, u_ref, d_ref, ys_ref):
    xb = xs_ref[...].astype(jnp.bfloat16)
    g = lax.dot_general(xb, g_ref[0], (((1,), (1,)), ((), ())),
                        preferred_element_type=jnp.float32)
    u = lax.dot_general(xb, u_ref[0], (((1,), (1,)), ((), ())),
                        preferred_element_type=jnp.float32)
    h = (g * jax.nn.sigmoid(g)) * u
    hd = lax.dot_general(h.astype(jnp.bfloat16), d_ref[0],
                         (((1,), (1,)), ((), ())),
                         preferred_element_type=jnp.float32)
    ys_ref[...] = hd


def _shared_body(xb_ref, g_ref, u_ref, d_ref, out_ref):
    h_idx = pl.program_id(1)
    xb = xb_ref[...]
    g = lax.dot_general(xb, g_ref[0], (((1,), (1,)), ((), ())),
                        preferred_element_type=jnp.float32)
    u = lax.dot_general(xb, u_ref[0], (((1,), (1,)), ((), ())),
                        preferred_element_type=jnp.float32)
    h = (g * jax.nn.sigmoid(g)) * u
    hd = lax.dot_general(h.astype(jnp.bfloat16), d_ref[0],
                         (((1,), (1,)), ((), ())),
                         preferred_element_type=jnp.float32)

    @pl.when(h_idx == 0)
    def _():
        out_ref[...] = hd

    @pl.when(h_idx > 0)
    def _():
        out_ref[...] += hd


def _combine_body(sh_ref, g0_ref, g1_ref, w0_ref, w1_ref, out_ref):
    w0 = w0_ref[...][:, 0:1]
    w1 = w1_ref[...][:, 0:1]
    out_ref[...] = sh_ref[...] + w0 * g0_ref[...] + w1 * g1_ref[...]


def kernel(hidden_states, gate_w, eg, eu, ed, sg, su, sd):
    orig_shape = hidden_states.shape
    d = orig_shape[-1]
    x = hidden_states.reshape(-1, d)
    n = x.shape[0]
    e_num, dff = eg.shape[0], eg.shape[1]
    nsh = sg.shape[0] // dff
    topk = 2
    nslot = topk * n
    mpad = nslot + e_num * _BLK
    nblk = mpad // _BLK

    xb = x.astype(jnp.bfloat16)
    gwt = gate_w.T.astype(jnp.bfloat16)
    eg16 = eg.astype(jnp.bfloat16)
    eu16 = eu.astype(jnp.bfloat16)
    ed16 = ed.astype(jnp.bfloat16)
    sgs = sg.reshape(nsh, dff, d).astype(jnp.bfloat16)
    sus = su.reshape(nsh, dff, d).astype(jnp.bfloat16)
    sds = sd.reshape(d, nsh, dff).transpose(1, 0, 2).astype(jnp.bfloat16)

    tm1 = 1024
    e0, e1, w0, w1 = pl.pallas_call(
        _gate_body,
        grid=(n // tm1,),
        in_specs=[
            pl.BlockSpec((tm1, d), lambda m: (m, 0)),
            pl.BlockSpec((d, e_num), lambda m: (0, 0)),
        ],
        out_specs=[
            pl.BlockSpec((tm1 // 128, 128), lambda m: (m, 0)),
            pl.BlockSpec((tm1 // 128, 128), lambda m: (m, 0)),
            pl.BlockSpec((tm1, 128), lambda m: (m, 0)),
            pl.BlockSpec((tm1, 128), lambda m: (m, 0)),
        ],
        out_shape=[
            jax.ShapeDtypeStruct((n // 128, 128), jnp.int32),
            jax.ShapeDtypeStruct((n // 128, 128), jnp.int32),
            jax.ShapeDtypeStruct((n, 128), jnp.float32),
            jax.ShapeDtypeStruct((n, 128), jnp.float32),
        ],
    )(xb, gwt)

    dest, blk = pl.pallas_call(
        functools.partial(_disp_body, e_num=e_num),
        in_specs=[pl.BlockSpec((n // 128, 128), lambda: (0, 0)),
                  pl.BlockSpec((n // 128, 128), lambda: (0, 0))],
        out_specs=[pl.BlockSpec((nslot // 128, 128), lambda: (0, 0)),
                   pl.BlockSpec((8, 128), lambda: (0, 0))],
        out_shape=[
            jax.ShapeDtypeStruct((nslot // 128, 128), jnp.int32),
            jax.ShapeDtypeStruct((8, 128), jnp.int32),
        ],
    )(e0, e1)
    dst1 = dest.reshape(nslot)
    blk1 = blk.reshape(-1)[:nblk]
    tok1 = (lax.iota(jnp.int32, nslot) & (n - 1))

    mesh = plsc.VectorSubcoreMesh(core_axis_name="c", subcore_axis_name="s")
    chunk = 32
    slots_per = nslot // 32
    xs3 = pl.kernel(
        functools.partial(_permute_body, slots_per=slots_per, chunk=chunk),
        out_type=jax.ShapeDtypeStruct((mpad, d), jnp.float32),
        mesh=mesh,
        scratch_types=[
            pltpu.VMEM((chunk,), jnp.int32),
            pltpu.VMEM((chunk,), jnp.int32),
            pltpu.VMEM((chunk, d), jnp.float32),
            pltpu.SemaphoreType.DMA,
        ],
    )(x, tok1, dst1)
    xs = xs3

    ys = pl.pallas_call(
        _gmm_body,
        grid_spec=pltpu.PrefetchScalarGridSpec(
            num_scalar_prefetch=1,
            grid=(nblk,),
            in_specs=[
                pl.BlockSpec((_BLK, d), lambda b, sm: (b, 0)),
                pl.BlockSpec((1, dff, d), lambda b, sm: (sm[b], 0, 0)),
                pl.BlockSpec((1, dff, d), lambda b, sm: (sm[b], 0, 0)),
                pl.BlockSpec((1, d, dff), lambda b, sm: (sm[b], 0, 0)),
            ],
            out_specs=pl.BlockSpec((_BLK, d), lambda b, sm: (b, 0)),
        ),
        out_shape=jax.ShapeDtypeStruct((mpad, d), jnp.float32),
        compiler_params=pltpu.CompilerParams(
            dimension_semantics=("arbitrary",)),
    )(blk1, xs, eg16, eu16, ed16)

    gsl = pl.kernel(
        functools.partial(_collect_body, slots_per=slots_per, chunk=chunk),
        out_type=jax.ShapeDtypeStruct((nslot, d), jnp.float32),
        mesh=mesh,
        scratch_types=[
            pltpu.VMEM((chunk,), jnp.int32),
            pltpu.VMEM((chunk, d), jnp.float32),
            pltpu.SemaphoreType.DMA,
        ],
    )(ys, dst1)

    tm6 = 256
    sh = pl.pallas_call(
        _shared_body,
        grid=(n // tm6, nsh),
        in_specs=[
            pl.BlockSpec((tm6, d), lambda m, h: (m, 0)),
            pl.BlockSpec((1, dff, d), lambda m, h: (h, 0, 0)),
            pl.BlockSpec((1, dff, d), lambda m, h: (h, 0, 0)),
            pl.BlockSpec((1, d, dff), lambda m, h: (h, 0, 0)),
        ],
        out_specs=pl.BlockSpec((tm6, d), lambda m, h: (m, 0)),
        out_shape=jax.ShapeDtypeStruct((n, d), jnp.float32),
        compiler_params=pltpu.CompilerParams(
            dimension_semantics=("arbitrary", "arbitrary")),
    )(xb, sgs, sus, sds)

    tm7 = 512
    out = pl.pallas_call(
        _combine_body,
        grid=(n // tm7,),
        in_specs=[
            pl.BlockSpec((tm7, d), lambda m: (m, 0)),
            pl.BlockSpec((tm7, d), lambda m: (m, 0)),
            pl.BlockSpec((tm7, d), lambda m: (m + n // tm7, 0)),
            pl.BlockSpec((tm7, 128), lambda m: (m, 0)),
            pl.BlockSpec((tm7, 128), lambda m: (m, 0)),
        ],
        out_specs=pl.BlockSpec((tm7, d), lambda m: (m, 0)),
        out_shape=jax.ShapeDtypeStruct((n, d), jnp.float32),
    )(sh, gsl, gsl, w0, w1)
    return out.reshape(orig_shape)


# speedup vs baseline: 1.0282x; 1.0282x over previous
import functools

import jax
import jax.numpy as jnp
from jax import lax
from jax.experimental import pallas as pl
from jax.experimental.pallas import tpu as pltpu
from jax.experimental.pallas import tpu_sc as plsc

_BLK = 256


def _gate_body(xb_ref, gwt_ref, e0_ref, e1_ref, w0_ref, w1_ref):
    logits = lax.dot_general(xb_ref[...], gwt_ref[...], (((1,), (0,)), ((), ())),
                             preferred_element_type=jnp.float32)
    mx = jnp.max(logits, axis=1, keepdims=True)
    ex = jnp.exp(logits - mx)
    p = ex / jnp.sum(ex, axis=1, keepdims=True)
    tm, ne = p.shape
    a = p[:, :, None]
    b = p[:, None, :]
    ii = lax.broadcasted_iota(jnp.int32, (tm, ne, ne), 1)
    jj = lax.broadcasted_iota(jnp.int32, (tm, ne, ne), 2)
    beats = (b > a) | ((b == a) & (jj < ii))
    rank = jnp.sum(beats.astype(jnp.int32), axis=2)
    lane = lax.broadcasted_iota(jnp.int32, (tm, ne), 1)
    for k, (eref, wref) in enumerate(((e0_ref, w0_ref), (e1_ref, w1_ref))):
        sel = rank == k
        idx = jnp.sum(jnp.where(sel, lane, 0), axis=1)
        wv = jnp.sum(jnp.where(sel, p, 0.0), axis=1)
        eref[...] = idx.reshape(eref.shape)
        wref[...] = jnp.broadcast_to(wv[:, None], wref.shape)


def _disp_body(e0_ref, e1_ref, dest_ref, blk_ref, *, e_num):
    eid = jnp.concatenate([e0_ref[...], e1_ref[...]], axis=0)
    rows, lanes = eid.shape
    ci = lax.broadcasted_iota(jnp.int32, (lanes, lanes), 0)
    cj = lax.broadcasted_iota(jnp.int32, (lanes, lanes), 1)
    slt = (ci < cj).astype(jnp.bfloat16)
    ri = lax.broadcasted_iota(jnp.int32, (rows, rows), 0)
    rj = lax.broadcasted_iota(jnp.int32, (rows, rows), 1)
    ltr = (rj < ri).astype(jnp.bfloat16)
    ones = jnp.ones((lanes, lanes), jnp.bfloat16)
    dest = jnp.zeros((rows, lanes), jnp.float32)
    brows, blanes = blk_ref.shape
    bi = lax.broadcasted_iota(jnp.int32, (brows, blanes), 1).astype(jnp.float32)
    blkv = jnp.full((brows, blanes), -1.0, jnp.float32)
    rowbase = jnp.float32(0.0)
    blkbase = jnp.float32(0.0)
    for e in range(e_num):
        m = (eid == e).astype(jnp.bfloat16)
        lane_excl = lax.dot_general(m, slt, (((1,), (0,)), ((), ())),
                                    preferred_element_type=jnp.float32)
        rowsum = lax.dot_general(m, ones, (((1,), (0,)), ((), ())),
                                 preferred_element_type=jnp.float32)
        row_excl = lax.dot_general(ltr, rowsum.astype(jnp.bfloat16),
                                   (((1,), (0,)), ((), ())),
                                   preferred_element_type=jnp.float32)
        prefix = lane_excl + row_excl
        cnt = jnp.sum(rowsum[:, 0:1])
        dest = dest + jnp.where(eid == e, rowbase + prefix, 0.0)
        blkv = blkv + (bi >= blkbase).astype(jnp.float32)
        nb = jnp.ceil(cnt / _BLK)
        blkbase = blkbase + nb
        rowbase = rowbase + nb * _BLK
    blkv = blkv + (bi >= blkbase).astype(jnp.float32)
    dest_ref[...] = dest.astype(jnp.int32)
    blk_ref[...] = blkv.astype(jnp.int32)


def _permute_body(x3, tok, dst, xs_o, v_tok, v_dst, v_rows, sem,
                  *, slots_per, chunk):
    c = lax.axis_index("c")
    s = lax.axis_index("s")
    wid = s * 2 + c
    for ch in range(slots_per // chunk):
        base = wid * slots_per + ch * chunk
        pltpu.sync_copy(tok.at[pl.ds(base, chunk)], v_tok)
        pltpu.sync_copy(dst.at[pl.ds(base, chunk)], v_dst)
        pltpu.async_copy(x3.at[v_tok], v_rows, sem).wait()
        pltpu.async_copy(v_rows, xs_o.at[v_dst], sem).wait()


def _collect_body(ys3, dst, gsl_o, v_dst, v_rows, sem, *, slots_per, chunk):
    c = lax.axis_index("c")
    s = lax.axis_index("s")
    wid = s * 2 + c
    for ch in range(slots_per // chunk):
        base = wid * slots_per + ch * chunk
        pltpu.sync_copy(dst.at[pl.ds(base, chunk)], v_dst)
        pltpu.async_copy(ys3.at[v_dst], v_rows, sem).wait()
        pltpu.sync_copy(v_rows, gsl_o.at[pl.ds(base, chunk)])


def _gmm_body(blk_sm, xs_ref, g_ref, u_ref, d_ref, ys_ref, *, e_num):
    @pl.when(blk_sm[pl.program_id(0)] < e_num)
    def _():
        xb = xs_ref[...].astype(jnp.bfloat16)
        g = lax.dot_general(xb, g_ref[0], (((1,), (1,)), ((), ())),
                            preferred_element_type=jnp.float32)
        u = lax.dot_general(xb, u_ref[0], (((1,), (1,)), ((), ())),
                            preferred_element_type=jnp.float32)
        h = (g * jax.nn.sigmoid(g)) * u
        hd = lax.dot_general(h.astype(jnp.bfloat16), d_ref[0],
                             (((1,), (1,)), ((), ())),
                             preferred_element_type=jnp.float32)
        ys_ref[...] = hd


def _combine_body(xb_ref, g0_ref, g1_ref, w0_ref, w1_ref,
                  g_ref, u_ref, d_ref, out_ref):
    h_idx = pl.program_id(1)
    xb = xb_ref[...]
    g = lax.dot_general(xb, g_ref[0], (((1,), (1,)), ((), ())),
                        preferred_element_type=jnp.float32)
    u = lax.dot_general(xb, u_ref[0], (((1,), (1,)), ((), ())),
                        preferred_element_type=jnp.float32)
    h = (g * jax.nn.sigmoid(g)) * u
    hd = lax.dot_general(h.astype(jnp.bfloat16), d_ref[0],
                         (((1,), (1,)), ((), ())),
                         preferred_element_type=jnp.float32)

    @pl.when(h_idx == 0)
    def _():
        w0 = w0_ref[...][:, 0:1]
        w1 = w1_ref[...][:, 0:1]
        out_ref[...] = w0 * g0_ref[...] + w1 * g1_ref[...] + hd

    @pl.when(h_idx > 0)
    def _():
        out_ref[...] += hd


def kernel(hidden_states, gate_w, eg, eu, ed, sg, su, sd):
    orig_shape = hidden_states.shape
    d = orig_shape[-1]
    x = hidden_states.reshape(-1, d)
    n = x.shape[0]
    e_num, dff = eg.shape[0], eg.shape[1]
    nsh = sg.shape[0] // dff
    topk = 2
    nslot = topk * n
    mpad = nslot + e_num * _BLK
    nblk = mpad // _BLK

    xb = x.astype(jnp.bfloat16)
    gwt = gate_w.T.astype(jnp.bfloat16)
    eg16 = eg.astype(jnp.bfloat16)
    eu16 = eu.astype(jnp.bfloat16)
    ed16 = ed.astype(jnp.bfloat16)
    sgs = sg.reshape(nsh, dff, d).astype(jnp.bfloat16)
    sus = su.reshape(nsh, dff, d).astype(jnp.bfloat16)
    sds = sd.reshape(d, nsh, dff).transpose(1, 0, 2).astype(jnp.bfloat16)

    tm1 = 1024
    e0, e1, w0, w1 = pl.pallas_call(
        _gate_body,
        grid=(n // tm1,),
        in_specs=[
            pl.BlockSpec((tm1, d), lambda m: (m, 0)),
            pl.BlockSpec((d, e_num), lambda m: (0, 0)),
        ],
        out_specs=[
            pl.BlockSpec((tm1 // 128, 128), lambda m: (m, 0)),
            pl.BlockSpec((tm1 // 128, 128), lambda m: (m, 0)),
            pl.BlockSpec((tm1, 128), lambda m: (m, 0)),
            pl.BlockSpec((tm1, 128), lambda m: (m, 0)),
        ],
        out_shape=[
            jax.ShapeDtypeStruct((n // 128, 128), jnp.int32),
            jax.ShapeDtypeStruct((n // 128, 128), jnp.int32),
            jax.ShapeDtypeStruct((n, 128), jnp.float32),
            jax.ShapeDtypeStruct((n, 128), jnp.float32),
        ],
    )(xb, gwt)

    dest, blk = pl.pallas_call(
        functools.partial(_disp_body, e_num=e_num),
        in_specs=[pl.BlockSpec((n // 128, 128), lambda: (0, 0)),
                  pl.BlockSpec((n // 128, 128), lambda: (0, 0))],
        out_specs=[pl.BlockSpec((nslot // 128, 128), lambda: (0, 0)),
                   pl.BlockSpec((8, 128), lambda: (0, 0))],
        out_shape=[
            jax.ShapeDtypeStruct((nslot // 128, 128), jnp.int32),
            jax.ShapeDtypeStruct((8, 128), jnp.int32),
        ],
    )(e0, e1)
    dst1 = dest.reshape(nslot)
    blk1 = blk.reshape(-1)[:nblk]
    tok1 = (lax.iota(jnp.int32, nslot) & (n - 1))

    mesh = plsc.VectorSubcoreMesh(core_axis_name="c", subcore_axis_name="s")
    chunk = 32
    slots_per = nslot // 32
    xs3 = pl.kernel(
        functools.partial(_permute_body, slots_per=slots_per, chunk=chunk),
        out_type=jax.ShapeDtypeStruct((mpad, d), jnp.float32),
        mesh=mesh,
        scratch_types=[
            pltpu.VMEM((chunk,), jnp.int32),
            pltpu.VMEM((chunk,), jnp.int32),
            pltpu.VMEM((chunk, d), jnp.float32),
            pltpu.SemaphoreType.DMA,
        ],
    )(x, tok1, dst1)
    xs = xs3

    wix = lambda b, sm: (jnp.minimum(sm[b], e_num - 1), 0, 0)
    ys = pl.pallas_call(
        functools.partial(_gmm_body, e_num=e_num),
        grid_spec=pltpu.PrefetchScalarGridSpec(
            num_scalar_prefetch=1,
            grid=(nblk,),
            in_specs=[
                pl.BlockSpec((_BLK, d), lambda b, sm: (b, 0)),
                pl.BlockSpec((1, dff, d), wix),
                pl.BlockSpec((1, dff, d), wix),
                pl.BlockSpec((1, d, dff), wix),
            ],
            out_specs=pl.BlockSpec((_BLK, d), lambda b, sm: (b, 0)),
        ),
        out_shape=jax.ShapeDtypeStruct((mpad, d), jnp.float32),
        compiler_params=pltpu.CompilerParams(
            dimension_semantics=("arbitrary",)),
    )(blk1, xs, eg16, eu16, ed16)

    gsl = pl.kernel(
        functools.partial(_collect_body, slots_per=slots_per, chunk=chunk),
        out_type=jax.ShapeDtypeStruct((nslot, d), jnp.float32),
        mesh=mesh,
        scratch_types=[
            pltpu.VMEM((chunk,), jnp.int32),
            pltpu.VMEM((chunk, d), jnp.float32),
            pltpu.SemaphoreType.DMA,
        ],
    )(ys, dst1)

    tm6 = 256
    out = pl.pallas_call(
        _combine_body,
        grid=(n // tm6, nsh),
        in_specs=[
            pl.BlockSpec((tm6, d), lambda m, h: (m, 0)),
            pl.BlockSpec((tm6, d), lambda m, h: (m, 0)),
            pl.BlockSpec((tm6, d), lambda m, h: (m + n // tm6, 0)),
            pl.BlockSpec((tm6, 128), lambda m, h: (m, 0)),
            pl.BlockSpec((tm6, 128), lambda m, h: (m, 0)),
            pl.BlockSpec((1, dff, d), lambda m, h: (h, 0, 0)),
            pl.BlockSpec((1, dff, d), lambda m, h: (h, 0, 0)),
            pl.BlockSpec((1, d, dff), lambda m, h: (h, 0, 0)),
        ],
        out_specs=pl.BlockSpec((tm6, d), lambda m, h: (m, 0)),
        out_shape=jax.ShapeDtypeStruct((n, d), jnp.float32),
        compiler_params=pltpu.CompilerParams(
            dimension_semantics=("arbitrary", "arbitrary")),
    )(xb, gsl, gsl, w0, w1, sgs, sus, sds)
    return out.reshape(orig_shape)
